# baseline (speedup 1.0000x reference)
import functools

import jax
import jax.numpy as jnp
from jax import lax
from jax.experimental import pallas as pl
from jax.experimental.pallas import tpu as pltpu
from jax.experimental.pallas import tpu_sc as plsc

N = 10000
E = 320000
F_IN = 128
DIM = 128
C = 64
EPS = 1e-5

NC = 2
NS = 16
NW = NC * NS
K = 128

NPAD = 10112
ROWS_PER_SC = NPAD // NS
DUMMY = 10008
NCH0 = 58
NCH1 = 102
TOTCH = NS * (NCH0 + NCH1)
EPAD = TOTCH * K

_ZR = 64
_NBUF = 4


def _sc_mesh():
    return plsc.VectorSubcoreMesh(
        core_axis_name="c", subcore_axis_name="s", num_cores=NC, num_subcores=NS
    )


def _fill_2d(ref, rows, width, value):
    v = jnp.full((16,), value, jnp.float32)

    def body(i, _):
        def inner(j, _):
            ref[i, pl.ds(j * 16, 16)] = v
            return 0

        return lax.fori_loop(0, width // 16, inner, 0)

    lax.fori_loop(0, rows, body, 0)


def _zero_shared(zbuf, acc, row0, nrows, width):
    nfull = nrows // _ZR
    rem = nrows - nfull * _ZR

    def body(i, _):
        pltpu.sync_copy(zbuf, acc.at[pl.ds(row0 + i * _ZR, _ZR), :])
        return 0

    lax.fori_loop(0, nfull, body, 0)
    if rem:
        pltpu.sync_copy(
            zbuf.at[pl.ds(0, rem), :], acc.at[pl.ds(row0 + nfull * _ZR, rem), :]
        )


def _count_body(dst_hbm, out_hbm, dst_v, ones_v, zbuf_v, acc_sh, sem):
    c = lax.axis_index("c")
    s = lax.axis_index("s")
    wid = c * NS + s
    _fill_2d(zbuf_v, _ZR, DIM, 0.0)
    _fill_2d(ones_v, K, DIM, 1.0)
    _zero_shared(zbuf_v, acc_sh, s * ROWS_PER_SC, ROWS_PER_SC, DIM)
    plsc.subcore_barrier()

    nch = jnp.where(c == 0, NCH0, NCH1)
    base = (c * NS * NCH0 + s * nch) * K

    def chunk(t, _):
        pltpu.sync_copy(dst_hbm.at[pl.ds(base + t * K, K)], dst_v)
        pltpu.sync_copy(ones_v, acc_sh.at[dst_v], add=True)
        return 0

    lax.fori_loop(0, nch, chunk, 0)
    plsc.subcore_barrier()
    pltpu.sync_copy(
        acc_sh.at[pl.ds(s * ROWS_PER_SC, ROWS_PER_SC), :],
        out_hbm.at[c, pl.ds(s * ROWS_PER_SC, ROWS_PER_SC), :],
    )


def _agg_body(u_hbm, src_hbm, dst_hbm, out_hbm, src_v, dst_v,
              rows0, zbuf_v, acc_sh, g0):
    c = lax.axis_index("c")
    s = lax.axis_index("s")
    wid = c * NS + s
    _fill_2d(zbuf_v, _ZR, DIM, 0.0)
    _zero_shared(zbuf_v, acc_sh, s * ROWS_PER_SC, ROWS_PER_SC, DIM)
    plsc.subcore_barrier()

    nch = jnp.where(c == 0, NCH0, NCH1)
    base = (c * NS * NCH0 + s * nch) * K

    def body(t, _):
        off = base + t * K
        pltpu.sync_copy(src_hbm.at[pl.ds(off, K)], src_v)
        pltpu.sync_copy(dst_hbm.at[pl.ds(off, K)], dst_v)
        pltpu.async_copy(u_hbm.at[src_v], rows0, g0).wait()
        pltpu.sync_copy(rows0, acc_sh.at[dst_v], add=True)
        return 0

    lax.fori_loop(0, nch, body, 0)
    plsc.subcore_barrier()
    pltpu.sync_copy(
        acc_sh.at[pl.ds(s * ROWS_PER_SC, ROWS_PER_SC), :],
        out_hbm.at[c, pl.ds(s * ROWS_PER_SC, ROWS_PER_SC), :],
    )


@jax.jit
def _sc_count(dst):
    return pl.kernel(
        _count_body,
        out_type=jax.ShapeDtypeStruct((NC, NPAD, DIM), jnp.float32),
        mesh=_sc_mesh(),
        scratch_types=[
            pltpu.VMEM((K,), jnp.int32),
            pltpu.VMEM((K, DIM), jnp.float32),
            pltpu.VMEM((_ZR, DIM), jnp.float32),
            pltpu.VMEM_SHARED((NPAD, DIM), jnp.float32),
            pltpu.SemaphoreType.DMA,
        ],
    )(dst)


@jax.jit
def _sc_agg(u, src, dst):
    return pl.kernel(
        _agg_body,
        out_type=jax.ShapeDtypeStruct((NC, NPAD, DIM), jnp.float32),
        mesh=_sc_mesh(),
        scratch_types=[
            pltpu.VMEM((K,), jnp.int32),
            pltpu.VMEM((K,), jnp.int32),
            pltpu.VMEM((K, DIM), jnp.float32),
            pltpu.VMEM((_ZR, DIM), jnp.float32),
            pltpu.VMEM_SHARED((NPAD, DIM), jnp.float32),
            pltpu.SemaphoreType.DMA,
        ],
    )(u, src, dst)


GB = 8
R = NPAD // GB
_HI = lax.Precision.HIGHEST


def _row_mask(g):
    rowid = lax.broadcasted_iota(jnp.int32, (R, 1), 0) + g * R
    return jnp.where(rowid < N, 1.0, 0.0).astype(jnp.float32)


def _prep_body(x_ref, cnt_ref, w0_ref, fcw0_ref, fcb_ref, dis_ref, h_ref, u_ref, y_ref):
    cnt = cnt_ref[0, :, 0:1] + cnt_ref[1, :, 0:1]
    dis = lax.rsqrt(cnt + 2.0)
    x = x_ref[...]
    h = jnp.dot(x, w0_ref[...], preferred_element_type=jnp.float32, precision=_HI)
    dis_ref[...] = dis
    h_ref[...] = h
    u_ref[...] = dis * h
    y_ref[...] = jnp.dot(x, fcw0_ref[...], preferred_element_type=jnp.float32, precision=_HI) + fcb_ref[...]


@jax.jit
def _tc_prep(x, cnt, W0, fcW0, fcb_sum):
    return pl.pallas_call(
        _prep_body,
        grid=(GB,),
        in_specs=[
            pl.BlockSpec((R, DIM), lambda g: (g, 0)),
            pl.BlockSpec((NC, R, DIM), lambda g: (0, g, 0)),
            pl.BlockSpec((DIM, DIM), lambda g: (0, 0)),
            pl.BlockSpec((DIM, C), lambda g: (0, 0)),
            pl.BlockSpec((1, C), lambda g: (0, 0)),
        ],
        out_specs=[
            pl.BlockSpec((R, 1), lambda g: (g, 0)),
            pl.BlockSpec((R, DIM), lambda g: (g, 0)),
            pl.BlockSpec((R, DIM), lambda g: (g, 0)),
            pl.BlockSpec((R, C), lambda g: (g, 0)),
        ],
        out_shape=[
            jax.ShapeDtypeStruct((NPAD, 1), jnp.float32),
            jax.ShapeDtypeStruct((NPAD, DIM), jnp.float32),
            jax.ShapeDtypeStruct((NPAD, DIM), jnp.float32),
            jax.ShapeDtypeStruct((NPAD, C), jnp.float32),
        ],
    )(x, cnt, W0, fcW0, fcb_sum)


def _stats_body(parts_ref, h_ref, dis_ref, b_ref, o_ref, sums_ref):
    g = pl.program_id(0)
    dis = dis_ref[...]
    tmp = parts_ref[0] + parts_ref[1]
    o = (dis * tmp + (2.0 * dis * dis) * h_ref[...] + b_ref[...]) * _row_mask(g)
    o_ref[...] = o
    s0 = jnp.sum(o, axis=0, keepdims=True)
    s1 = jnp.sum(o * o, axis=0, keepdims=True)
    blk = jnp.concatenate([s0, s1, jnp.zeros((6, DIM), jnp.float32)], axis=0)

    @pl.when(g == 0)
    def _():
        sums_ref[...] = blk

    @pl.when(g != 0)
    def _():
        sums_ref[...] = sums_ref[...] + blk


@jax.jit
def _tc_stats(parts, h, dis, b):
    return pl.pallas_call(
        _stats_body,
        grid=(GB,),
        in_specs=[
            pl.BlockSpec((NC, R, DIM), lambda g: (0, g, 0)),
            pl.BlockSpec((R, DIM), lambda g: (g, 0)),
            pl.BlockSpec((R, 1), lambda g: (g, 0)),
            pl.BlockSpec((1, DIM), lambda g: (0, 0)),
        ],
        out_specs=[
            pl.BlockSpec((R, DIM), lambda g: (g, 0)),
            pl.BlockSpec((8, DIM), lambda g: (0, 0)),
        ],
        out_shape=[
            jax.ShapeDtypeStruct((NPAD, DIM), jnp.float32),
            jax.ShapeDtypeStruct((8, DIM), jnp.float32),
        ],
    )(parts, h, dis, b)


def _bn_relu_block(o_ref, sums_ref, g_ref, beta_ref, gidx):
    m = sums_ref[0:1, :] * (1.0 / N)
    s2 = sums_ref[1:2, :] * (1.0 / N)
    v = s2 - m * m
    hb = (o_ref[...] - m) * lax.rsqrt(v + EPS) * g_ref[...] + beta_ref[...]
    return jnp.maximum(hb, 0.0) * _row_mask(gidx)


def _apply_body(o_ref, sums_ref, dis_ref, g_ref, beta_ref, w_ref, fcw_ref, y_ref,
                hn_ref, u_ref, yn_ref):
    gidx = pl.program_id(0)
    hb = _bn_relu_block(o_ref, sums_ref, g_ref, beta_ref, gidx)
    hw = jnp.dot(hb, w_ref[...], preferred_element_type=jnp.float32, precision=_HI)
    hn_ref[...] = hw
    u_ref[...] = dis_ref[...] * hw
    yn_ref[...] = y_ref[...] + jnp.dot(hb, fcw_ref[...], preferred_element_type=jnp.float32, precision=_HI)


@jax.jit
def _tc_apply(o, sums, dis, g, beta, W_next, fcW_next, y):
    return pl.pallas_call(
        _apply_body,
        grid=(GB,),
        in_specs=[
            pl.BlockSpec((R, DIM), lambda g: (g, 0)),
            pl.BlockSpec((8, DIM), lambda g: (0, 0)),
            pl.BlockSpec((R, 1), lambda g: (g, 0)),
            pl.BlockSpec((1, DIM), lambda g: (0, 0)),
            pl.BlockSpec((1, DIM), lambda g: (0, 0)),
            pl.BlockSpec((DIM, DIM), lambda g: (0, 0)),
            pl.BlockSpec((DIM, C), lambda g: (0, 0)),
            pl.BlockSpec((R, C), lambda g: (g, 0)),
        ],
        out_specs=[
            pl.BlockSpec((R, DIM), lambda g: (g, 0)),
            pl.BlockSpec((R, DIM), lambda g: (g, 0)),
            pl.BlockSpec((R, C), lambda g: (g, 0)),
        ],
        out_shape=[
            jax.ShapeDtypeStruct((NPAD, DIM), jnp.float32),
            jax.ShapeDtypeStruct((NPAD, DIM), jnp.float32),
            jax.ShapeDtypeStruct((NPAD, C), jnp.float32),
        ],
    )(o, sums, dis, g, beta, W_next, fcW_next, y)


def _final_body(o_ref, sums_ref, g_ref, beta_ref, fcw_ref, y_ref, out_ref):
    gidx = pl.program_id(0)
    hb = _bn_relu_block(o_ref, sums_ref, g_ref, beta_ref, gidx)
    y = y_ref[...] + jnp.dot(hb, fcw_ref[...], preferred_element_type=jnp.float32, precision=_HI)
    z = y - jnp.max(y, axis=-1, keepdims=True)
    out_ref[...] = z - jnp.log(jnp.sum(jnp.exp(z), axis=-1, keepdims=True))


@jax.jit
def _tc_final(o, sums, g, beta, fcW, y):
    return pl.pallas_call(
        _final_body,
        grid=(GB,),
        in_specs=[
            pl.BlockSpec((R, DIM), lambda g: (g, 0)),
            pl.BlockSpec((8, DIM), lambda g: (0, 0)),
            pl.BlockSpec((1, DIM), lambda g: (0, 0)),
            pl.BlockSpec((1, DIM), lambda g: (0, 0)),
            pl.BlockSpec((DIM, C), lambda g: (0, 0)),
            pl.BlockSpec((R, C), lambda g: (g, 0)),
        ],
        out_specs=pl.BlockSpec((R, C), lambda g: (g, 0)),
        out_shape=jax.ShapeDtypeStruct((NPAD, C), jnp.float32),
    )(o, sums, g, beta, fcW, y)


def kernel(x, edge_index, W0, b0, W1, b1, W2, b2, W3, b3,
           g0, beta0, g1, beta1, g2, beta2, g3, beta3,
           fcW0, fcb0, fcW1, fcb1, fcW2, fcb2, fcW3, fcb3, fcW4, fcb4):
    ei = edge_index.astype(jnp.int32)
    pad = jnp.full((EPAD - E,), DUMMY, jnp.int32)
    src = jnp.concatenate([ei[0], pad])
    dst = jnp.concatenate([ei[1], pad])
    xp = jnp.concatenate([x, jnp.zeros((NPAD - N, F_IN), jnp.float32)], axis=0)

    fcb_sum = (fcb0 + fcb1 + fcb2 + fcb3 + fcb4).reshape(1, C)
    Ws = [W1, W2, W3]
    bs = [b0.reshape(1, DIM), b1.reshape(1, DIM), b2.reshape(1, DIM), b3.reshape(1, DIM)]
    gs = [g0.reshape(1, DIM), g1.reshape(1, DIM), g2.reshape(1, DIM), g3.reshape(1, DIM)]
    betas = [beta0.reshape(1, DIM), beta1.reshape(1, DIM), beta2.reshape(1, DIM), beta3.reshape(1, DIM)]
    fcWs = [fcW1, fcW2, fcW3, fcW4]

    cnt = _sc_count(dst)
    dis, h, u, y = _tc_prep(xp, cnt, W0, fcW0, fcb_sum)
    for i in range(3):
        parts = _sc_agg(u, src, dst)
        o, sums = _tc_stats(parts, h, dis, bs[i])
        h, u, y = _tc_apply(o, sums, dis, gs[i], betas[i], Ws[i], fcWs[i], y)
    parts = _sc_agg(u, src, dst)
    o, sums = _tc_stats(parts, h, dis, bs[3])
    out = _tc_final(o, sums, gs[3], betas[3], fcWs[3], y)
    return out[:N]


# speedup vs baseline: 1.1900x; 1.1900x over previous
import functools

import jax
import jax.numpy as jnp
from jax import lax
from jax.experimental import pallas as pl
from jax.experimental.pallas import tpu as pltpu
from jax.experimental.pallas import tpu_sc as plsc

N = 10000
E = 320000
F_IN = 128
DIM = 128
C = 64
EPS = 1e-5

NC = 2
NS = 16
NW = NC * NS
K = 128

NPAD = 10112
ROWS_PER_SC = NPAD // NS
DUMMY = 10008
NCH0 = 102
NCH1 = 58
TOTCH = NS * (NCH0 + NCH1)
EPAD = TOTCH * K

_ZR = 64
_NBUF = 4


def _sc_mesh():
    return plsc.VectorSubcoreMesh(
        core_axis_name="c", subcore_axis_name="s", num_cores=NC, num_subcores=NS
    )


def _fill_2d(ref, rows, width, value):
    v = jnp.full((16,), value, jnp.float32)

    def body(i, _):
        def inner(j, _):
            ref[i, pl.ds(j * 16, 16)] = v
            return 0

        return lax.fori_loop(0, width // 16, inner, 0)

    lax.fori_loop(0, rows, body, 0)


def _zero_shared(zbuf, acc, row0, nrows, width):
    nfull = nrows // _ZR
    rem = nrows - nfull * _ZR

    def body(i, _):
        pltpu.sync_copy(zbuf, acc.at[pl.ds(row0 + i * _ZR, _ZR), :])
        return 0

    lax.fori_loop(0, nfull, body, 0)
    if rem:
        pltpu.sync_copy(
            zbuf.at[pl.ds(0, rem), :], acc.at[pl.ds(row0 + nfull * _ZR, rem), :]
        )


def _count_body(dst_hbm, out_hbm, dst_v, ones_v, zbuf_v, acc_sh, sem):
    c = lax.axis_index("c")
    s = lax.axis_index("s")
    wid = c * NS + s
    _fill_2d(zbuf_v, _ZR, DIM, 0.0)
    _fill_2d(ones_v, K, DIM, 1.0)
    _zero_shared(zbuf_v, acc_sh, s * ROWS_PER_SC, ROWS_PER_SC, DIM)
    plsc.subcore_barrier()

    nch = jnp.where(c == 0, NCH0, NCH1)
    base = (c * NS * NCH0 + s * nch) * K

    def chunk(t, _):
        pltpu.sync_copy(dst_hbm.at[pl.ds(base + t * K, K)], dst_v)
        pltpu.sync_copy(ones_v, acc_sh.at[dst_v], add=True)
        return 0

    lax.fori_loop(0, nch, chunk, 0)
    plsc.subcore_barrier()
    pltpu.sync_copy(
        acc_sh.at[pl.ds(s * ROWS_PER_SC, ROWS_PER_SC), :],
        out_hbm.at[c, pl.ds(s * ROWS_PER_SC, ROWS_PER_SC), :],
    )


def _agg_body(u_hbm, src_hbm, dst_hbm, out_hbm, src_v, dst_v,
              rows0, zbuf_v, acc_sh, g0):
    c = lax.axis_index("c")
    s = lax.axis_index("s")
    wid = c * NS + s
    _fill_2d(zbuf_v, _ZR, DIM, 0.0)
    _zero_shared(zbuf_v, acc_sh, s * ROWS_PER_SC, ROWS_PER_SC, DIM)
    plsc.subcore_barrier()

    nch = jnp.where(c == 0, NCH0, NCH1)
    base = (c * NS * NCH0 + s * nch) * K

    def body(t, _):
        off = base + t * K
        pltpu.sync_copy(src_hbm.at[pl.ds(off, K)], src_v)
        pltpu.sync_copy(dst_hbm.at[pl.ds(off, K)], dst_v)
        pltpu.async_copy(u_hbm.at[src_v], rows0, g0).wait()
        pltpu.sync_copy(rows0, acc_sh.at[dst_v], add=True)
        return 0

    lax.fori_loop(0, nch, body, 0)
    plsc.subcore_barrier()
    pltpu.sync_copy(
        acc_sh.at[pl.ds(s * ROWS_PER_SC, ROWS_PER_SC), :],
        out_hbm.at[c, pl.ds(s * ROWS_PER_SC, ROWS_PER_SC), :],
    )


@jax.jit
def _sc_count(dst):
    return pl.kernel(
        _count_body,
        out_type=jax.ShapeDtypeStruct((NC, NPAD, DIM), jnp.float32),
        mesh=_sc_mesh(),
        scratch_types=[
            pltpu.VMEM((K,), jnp.int32),
            pltpu.VMEM((K, DIM), jnp.float32),
            pltpu.VMEM((_ZR, DIM), jnp.float32),
            pltpu.VMEM_SHARED((NPAD, DIM), jnp.float32),
            pltpu.SemaphoreType.DMA,
        ],
    )(dst)


@jax.jit
def _sc_agg(u, src, dst):
    return pl.kernel(
        _agg_body,
        out_type=jax.ShapeDtypeStruct((NC, NPAD, DIM), jnp.float32),
        mesh=_sc_mesh(),
        scratch_types=[
            pltpu.VMEM((K,), jnp.int32),
            pltpu.VMEM((K,), jnp.int32),
            pltpu.VMEM((K, DIM), jnp.float32),
            pltpu.VMEM((_ZR, DIM), jnp.float32),
            pltpu.VMEM_SHARED((NPAD, DIM), jnp.float32),
            pltpu.SemaphoreType.DMA,
        ],
    )(u, src, dst)


GB = 8
R = NPAD // GB
_HI = lax.Precision.HIGHEST


def _row_mask(g):
    rowid = lax.broadcasted_iota(jnp.int32, (R, 1), 0) + g * R
    return jnp.where(rowid < N, 1.0, 0.0).astype(jnp.float32)


def _prep_body(x_ref, cnt_ref, w0_ref, fcw0_ref, fcb_ref, dis_ref, h_ref, u_ref, y_ref):
    cnt = cnt_ref[0, :, 0:1] + cnt_ref[1, :, 0:1]
    dis = lax.rsqrt(cnt + 2.0)
    x = x_ref[...]
    h = jnp.dot(x, w0_ref[...], preferred_element_type=jnp.float32, precision=_HI)
    dis_ref[...] = dis
    h_ref[...] = h
    u_ref[...] = dis * h
    y_ref[...] = jnp.dot(x, fcw0_ref[...], preferred_element_type=jnp.float32, precision=_HI) + fcb_ref[...]


@jax.jit
def _tc_prep(x, cnt, W0, fcW0, fcb_sum):
    return pl.pallas_call(
        _prep_body,
        grid=(GB,),
        in_specs=[
            pl.BlockSpec((R, DIM), lambda g: (g, 0)),
            pl.BlockSpec((NC, R, DIM), lambda g: (0, g, 0)),
            pl.BlockSpec((DIM, DIM), lambda g: (0, 0)),
            pl.BlockSpec((DIM, C), lambda g: (0, 0)),
            pl.BlockSpec((1, C), lambda g: (0, 0)),
        ],
        out_specs=[
            pl.BlockSpec((R, 1), lambda g: (g, 0)),
            pl.BlockSpec((R, DIM), lambda g: (g, 0)),
            pl.BlockSpec((R, DIM), lambda g: (g, 0)),
            pl.BlockSpec((R, C), lambda g: (g, 0)),
        ],
        out_shape=[
            jax.ShapeDtypeStruct((NPAD, 1), jnp.float32),
            jax.ShapeDtypeStruct((NPAD, DIM), jnp.float32),
            jax.ShapeDtypeStruct((NPAD, DIM), jnp.float32),
            jax.ShapeDtypeStruct((NPAD, C), jnp.float32),
        ],
    )(x, cnt, W0, fcW0, fcb_sum)


def _stats_body(parts_ref, h_ref, dis_ref, b_ref, o_ref, sums_ref):
    g = pl.program_id(0)
    dis = dis_ref[...]
    tmp = parts_ref[0] + parts_ref[1]
    o = (dis * tmp + (2.0 * dis * dis) * h_ref[...] + b_ref[...]) * _row_mask(g)
    o_ref[...] = o
    s0 = jnp.sum(o, axis=0, keepdims=True)
    s1 = jnp.sum(o * o, axis=0, keepdims=True)
    blk = jnp.concatenate([s0, s1, jnp.zeros((6, DIM), jnp.float32)], axis=0)

    @pl.when(g == 0)
    def _():
        sums_ref[...] = blk

    @pl.when(g != 0)
    def _():
        sums_ref[...] = sums_ref[...] + blk


@jax.jit
def _tc_stats(parts, h, dis, b):
    return pl.pallas_call(
        _stats_body,
        grid=(GB,),
        in_specs=[
            pl.BlockSpec((NC, R, DIM), lambda g: (0, g, 0)),
            pl.BlockSpec((R, DIM), lambda g: (g, 0)),
            pl.BlockSpec((R, 1), lambda g: (g, 0)),
            pl.BlockSpec((1, DIM), lambda g: (0, 0)),
        ],
        out_specs=[
            pl.BlockSpec((R, DIM), lambda g: (g, 0)),
            pl.BlockSpec((8, DIM), lambda g: (0, 0)),
        ],
        out_shape=[
            jax.ShapeDtypeStruct((NPAD, DIM), jnp.float32),
            jax.ShapeDtypeStruct((8, DIM), jnp.float32),
        ],
    )(parts, h, dis, b)


def _bn_relu_block(o_ref, sums_ref, g_ref, beta_ref, gidx):
    m = sums_ref[0:1, :] * (1.0 / N)
    s2 = sums_ref[1:2, :] * (1.0 / N)
    v = s2 - m * m
    hb = (o_ref[...] - m) * lax.rsqrt(v + EPS) * g_ref[...] + beta_ref[...]
    return jnp.maximum(hb, 0.0) * _row_mask(gidx)


def _apply_body(o_ref, sums_ref, dis_ref, g_ref, beta_ref, w_ref, fcw_ref, y_ref,
                hn_ref, u_ref, yn_ref):
    gidx = pl.program_id(0)
    hb = _bn_relu_block(o_ref, sums_ref, g_ref, beta_ref, gidx)
    hw = jnp.dot(hb, w_ref[...], preferred_element_type=jnp.float32, precision=_HI)
    hn_ref[...] = hw
    u_ref[...] = dis_ref[...] * hw
    yn_ref[...] = y_ref[...] + jnp.dot(hb, fcw_ref[...], preferred_element_type=jnp.float32, precision=_HI)


@jax.jit
def _tc_apply(o, sums, dis, g, beta, W_next, fcW_next, y):
    return pl.pallas_call(
        _apply_body,
        grid=(GB,),
        in_specs=[
            pl.BlockSpec((R, DIM), lambda g: (g, 0)),
            pl.BlockSpec((8, DIM), lambda g: (0, 0)),
            pl.BlockSpec((R, 1), lambda g: (g, 0)),
            pl.BlockSpec((1, DIM), lambda g: (0, 0)),
            pl.BlockSpec((1, DIM), lambda g: (0, 0)),
            pl.BlockSpec((DIM, DIM), lambda g: (0, 0)),
            pl.BlockSpec((DIM, C), lambda g: (0, 0)),
            pl.BlockSpec((R, C), lambda g: (g, 0)),
        ],
        out_specs=[
            pl.BlockSpec((R, DIM), lambda g: (g, 0)),
            pl.BlockSpec((R, DIM), lambda g: (g, 0)),
            pl.BlockSpec((R, C), lambda g: (g, 0)),
        ],
        out_shape=[
            jax.ShapeDtypeStruct((NPAD, DIM), jnp.float32),
            jax.ShapeDtypeStruct((NPAD, DIM), jnp.float32),
            jax.ShapeDtypeStruct((NPAD, C), jnp.float32),
        ],
    )(o, sums, dis, g, beta, W_next, fcW_next, y)


def _final_body(o_ref, sums_ref, g_ref, beta_ref, fcw_ref, y_ref, out_ref):
    gidx = pl.program_id(0)
    hb = _bn_relu_block(o_ref, sums_ref, g_ref, beta_ref, gidx)
    y = y_ref[...] + jnp.dot(hb, fcw_ref[...], preferred_element_type=jnp.float32, precision=_HI)
    z = y - jnp.max(y, axis=-1, keepdims=True)
    out_ref[...] = z - jnp.log(jnp.sum(jnp.exp(z), axis=-1, keepdims=True))


@jax.jit
def _tc_final(o, sums, g, beta, fcW, y):
    return pl.pallas_call(
        _final_body,
        grid=(GB,),
        in_specs=[
            pl.BlockSpec((R, DIM), lambda g: (g, 0)),
            pl.BlockSpec((8, DIM), lambda g: (0, 0)),
            pl.BlockSpec((1, DIM), lambda g: (0, 0)),
            pl.BlockSpec((1, DIM), lambda g: (0, 0)),
            pl.BlockSpec((DIM, C), lambda g: (0, 0)),
            pl.BlockSpec((R, C), lambda g: (g, 0)),
        ],
        out_specs=pl.BlockSpec((R, C), lambda g: (g, 0)),
        out_shape=jax.ShapeDtypeStruct((NPAD, C), jnp.float32),
    )(o, sums, g, beta, fcW, y)


def kernel(x, edge_index, W0, b0, W1, b1, W2, b2, W3, b3,
           g0, beta0, g1, beta1, g2, beta2, g3, beta3,
           fcW0, fcb0, fcW1, fcb1, fcW2, fcb2, fcW3, fcb3, fcW4, fcb4):
    ei = edge_index.astype(jnp.int32)
    pad = jnp.full((EPAD - E,), DUMMY, jnp.int32)
    src = jnp.concatenate([ei[0], pad])
    dst = jnp.concatenate([ei[1], pad])
    xp = jnp.concatenate([x, jnp.zeros((NPAD - N, F_IN), jnp.float32)], axis=0)

    fcb_sum = (fcb0 + fcb1 + fcb2 + fcb3 + fcb4).reshape(1, C)
    Ws = [W1, W2, W3]
    bs = [b0.reshape(1, DIM), b1.reshape(1, DIM), b2.reshape(1, DIM), b3.reshape(1, DIM)]
    gs = [g0.reshape(1, DIM), g1.reshape(1, DIM), g2.reshape(1, DIM), g3.reshape(1, DIM)]
    betas = [beta0.reshape(1, DIM), beta1.reshape(1, DIM), beta2.reshape(1, DIM), beta3.reshape(1, DIM)]
    fcWs = [fcW1, fcW2, fcW3, fcW4]

    cnt = _sc_count(dst)
    dis, h, u, y = _tc_prep(xp, cnt, W0, fcW0, fcb_sum)
    for i in range(3):
        parts = _sc_agg(u, src, dst)
        o, sums = _tc_stats(parts, h, dis, bs[i])
        h, u, y = _tc_apply(o, sums, dis, gs[i], betas[i], Ws[i], fcWs[i], y)
    parts = _sc_agg(u, src, dst)
    o, sums = _tc_stats(parts, h, dis, bs[3])
    out = _tc_final(o, sums, gs[3], betas[3], fcWs[3], y)
    return out[:N]


# speedup vs baseline: 1.2065x; 1.0139x over previous
import functools

import jax
import jax.numpy as jnp
from jax import lax
from jax.experimental import pallas as pl
from jax.experimental.pallas import tpu as pltpu
from jax.experimental.pallas import tpu_sc as plsc

N = 10000
E = 320000
F_IN = 128
DIM = 128
C = 64
EPS = 1e-5

NC = 2
NS = 16
NW = NC * NS
K = 128

NPAD = 10112
ROWS_PER_SC = NPAD // NS
DUMMY = 10008
NCH = 80
EPAD = NW * NCH * K

_ZR = 64
_NBUF = 4


def _sc_mesh():
    return plsc.VectorSubcoreMesh(
        core_axis_name="c", subcore_axis_name="s", num_cores=NC, num_subcores=NS
    )


def _fill_2d(ref, rows, width, value):
    v = jnp.full((16,), value, jnp.float32)

    def body(i, _):
        def inner(j, _):
            ref[i, pl.ds(j * 16, 16)] = v
            return 0

        return lax.fori_loop(0, width // 16, inner, 0)

    lax.fori_loop(0, rows, body, 0)


def _zero_shared(zbuf, acc, row0, nrows, width):
    nfull = nrows // _ZR
    rem = nrows - nfull * _ZR

    def body(i, _):
        pltpu.sync_copy(zbuf, acc.at[pl.ds(row0 + i * _ZR, _ZR), :])
        return 0

    lax.fori_loop(0, nfull, body, 0)
    if rem:
        pltpu.sync_copy(
            zbuf.at[pl.ds(0, rem), :], acc.at[pl.ds(row0 + nfull * _ZR, rem), :]
        )


def _count_body(dst_hbm, out_hbm, dst_v, ones_v, zbuf_v, acc_sh, sem):
    c = lax.axis_index("c")
    s = lax.axis_index("s")
    wid = c * NS + s
    _fill_2d(zbuf_v, _ZR, DIM, 0.0)
    _fill_2d(ones_v, K, DIM, 1.0)
    _zero_shared(zbuf_v, acc_sh, s * ROWS_PER_SC, ROWS_PER_SC, DIM)
    plsc.subcore_barrier()

    base = wid * NCH * K

    def chunk(t, _):
        pltpu.sync_copy(dst_hbm.at[pl.ds(base + t * K, K)], dst_v)
        pltpu.sync_copy(ones_v, acc_sh.at[dst_v], add=True)
        return 0

    lax.fori_loop(0, NCH, chunk, 0)
    plsc.subcore_barrier()
    pltpu.sync_copy(
        acc_sh.at[pl.ds(s * ROWS_PER_SC, ROWS_PER_SC), :],
        out_hbm.at[c, pl.ds(s * ROWS_PER_SC, ROWS_PER_SC), :],
    )


def _agg_body(u_hbm, src_hbm, dst_hbm, out_hbm, srcA, dstA, srcB, dstB,
              rows0, zbuf_v, acc_sh, gsem, isemA, isemB):
    c = lax.axis_index("c")
    s = lax.axis_index("s")
    wid = c * NS + s
    _fill_2d(zbuf_v, _ZR, DIM, 0.0)
    _zero_shared(zbuf_v, acc_sh, s * ROWS_PER_SC, ROWS_PER_SC, DIM)

    base = wid * NCH * K

    def idx_start(t, sref, dref, sem):
        pltpu.async_copy(src_hbm.at[pl.ds(base + t * K, K)], sref, sem)
        pltpu.async_copy(dst_hbm.at[pl.ds(base + t * K, K)], dref, sem)

    def idx_wait(t, sref, dref, sem):
        pltpu.make_async_copy(src_hbm.at[pl.ds(base + t * K, K)], sref, sem).wait()
        pltpu.make_async_copy(dst_hbm.at[pl.ds(base + t * K, K)], dref, sem).wait()

    idx_start(0, srcA, dstA, isemA)
    plsc.subcore_barrier()

    def do_chunk(sref, dref):
        pltpu.async_copy(u_hbm.at[sref], rows0, gsem).wait()
        pltpu.sync_copy(rows0, acc_sh.at[dref], add=True)

    def body(i, _):
        t0 = 2 * i
        idx_wait(t0, srcA, dstA, isemA)
        idx_start(t0 + 1, srcB, dstB, isemB)
        do_chunk(srcA, dstA)

        @pl.when(t0 + 2 < NCH)
        def _():
            idx_start(t0 + 2, srcA, dstA, isemA)

        idx_wait(t0 + 1, srcB, dstB, isemB)
        do_chunk(srcB, dstB)
        return 0

    lax.fori_loop(0, NCH // 2, body, 0)
    plsc.subcore_barrier()
    pltpu.sync_copy(
        acc_sh.at[pl.ds(s * ROWS_PER_SC, ROWS_PER_SC), :],
        out_hbm.at[c, pl.ds(s * ROWS_PER_SC, ROWS_PER_SC), :],
    )


@jax.jit
def _sc_count(dst):
    return pl.kernel(
        _count_body,
        out_type=jax.ShapeDtypeStruct((NC, NPAD, DIM), jnp.float32),
        mesh=_sc_mesh(),
        scratch_types=[
            pltpu.VMEM((K,), jnp.int32),
            pltpu.VMEM((K, DIM), jnp.float32),
            pltpu.VMEM((_ZR, DIM), jnp.float32),
            pltpu.VMEM_SHARED((NPAD, DIM), jnp.float32),
            pltpu.SemaphoreType.DMA,
        ],
    )(dst)


@jax.jit
def _sc_agg(u, src, dst):
    return pl.kernel(
        _agg_body,
        out_type=jax.ShapeDtypeStruct((NC, NPAD, DIM), jnp.float32),
        mesh=_sc_mesh(),
        scratch_types=[
            pltpu.VMEM((K,), jnp.int32),
            pltpu.VMEM((K,), jnp.int32),
            pltpu.VMEM((K,), jnp.int32),
            pltpu.VMEM((K,), jnp.int32),
            pltpu.VMEM((K, DIM), jnp.float32),
            pltpu.VMEM((_ZR, DIM), jnp.float32),
            pltpu.VMEM_SHARED((NPAD, DIM), jnp.float32),
            pltpu.SemaphoreType.DMA,
            pltpu.SemaphoreType.DMA,
            pltpu.SemaphoreType.DMA,
        ],
    )(u, src, dst)


GB = 8
R = NPAD // GB
_HI = lax.Precision.HIGHEST


def _row_mask(g):
    rowid = lax.broadcasted_iota(jnp.int32, (R, 1), 0) + g * R
    return jnp.where(rowid < N, 1.0, 0.0).astype(jnp.float32)


def _prep_body(x_ref, cnt_ref, w0_ref, fcw0_ref, fcb_ref, dis_ref, h_ref, u_ref, y_ref):
    cnt = cnt_ref[0, :, 0:1] + cnt_ref[1, :, 0:1]
    dis = lax.rsqrt(cnt + 2.0)
    x = x_ref[...]
    h = jnp.dot(x, w0_ref[...], preferred_element_type=jnp.float32, precision=_HI)
    dis_ref[...] = dis
    h_ref[...] = h
    u_ref[...] = dis * h
    y_ref[...] = jnp.dot(x, fcw0_ref[...], preferred_element_type=jnp.float32, precision=_HI) + fcb_ref[...]


@jax.jit
def _tc_prep(x, cnt, W0, fcW0, fcb_sum):
    return pl.pallas_call(
        _prep_body,
        grid=(GB,),
        in_specs=[
            pl.BlockSpec((R, DIM), lambda g: (g, 0)),
            pl.BlockSpec((NC, R, DIM), lambda g: (0, g, 0)),
            pl.BlockSpec((DIM, DIM), lambda g: (0, 0)),
            pl.BlockSpec((DIM, C), lambda g: (0, 0)),
            pl.BlockSpec((1, C), lambda g: (0, 0)),
        ],
        out_specs=[
            pl.BlockSpec((R, 1), lambda g: (g, 0)),
            pl.BlockSpec((R, DIM), lambda g: (g, 0)),
            pl.BlockSpec((R, DIM), lambda g: (g, 0)),
            pl.BlockSpec((R, C), lambda g: (g, 0)),
        ],
        out_shape=[
            jax.ShapeDtypeStruct((NPAD, 1), jnp.float32),
            jax.ShapeDtypeStruct((NPAD, DIM), jnp.float32),
            jax.ShapeDtypeStruct((NPAD, DIM), jnp.float32),
            jax.ShapeDtypeStruct((NPAD, C), jnp.float32),
        ],
    )(x, cnt, W0, fcW0, fcb_sum)


def _stats_body(parts_ref, h_ref, dis_ref, b_ref, o_ref, sums_ref):
    g = pl.program_id(0)
    dis = dis_ref[...]
    tmp = parts_ref[0] + parts_ref[1]
    o = (dis * tmp + (2.0 * dis * dis) * h_ref[...] + b_ref[...]) * _row_mask(g)
    o_ref[...] = o
    s0 = jnp.sum(o, axis=0, keepdims=True)
    s1 = jnp.sum(o * o, axis=0, keepdims=True)
    blk = jnp.concatenate([s0, s1, jnp.zeros((6, DIM), jnp.float32)], axis=0)

    @pl.when(g == 0)
    def _():
        sums_ref[...] = blk

    @pl.when(g != 0)
    def _():
        sums_ref[...] = sums_ref[...] + blk


@jax.jit
def _tc_stats(parts, h, dis, b):
    return pl.pallas_call(
        _stats_body,
        grid=(GB,),
        in_specs=[
            pl.BlockSpec((NC, R, DIM), lambda g: (0, g, 0)),
            pl.BlockSpec((R, DIM), lambda g: (g, 0)),
            pl.BlockSpec((R, 1), lambda g: (g, 0)),
            pl.BlockSpec((1, DIM), lambda g: (0, 0)),
        ],
        out_specs=[
            pl.BlockSpec((R, DIM), lambda g: (g, 0)),
            pl.BlockSpec((8, DIM), lambda g: (0, 0)),
        ],
        out_shape=[
            jax.ShapeDtypeStruct((NPAD, DIM), jnp.float32),
            jax.ShapeDtypeStruct((8, DIM), jnp.float32),
        ],
    )(parts, h, dis, b)


def _bn_relu_block(o_ref, sums_ref, g_ref, beta_ref, gidx):
    m = sums_ref[0:1, :] * (1.0 / N)
    s2 = sums_ref[1:2, :] * (1.0 / N)
    v = s2 - m * m
    hb = (o_ref[...] - m) * lax.rsqrt(v + EPS) * g_ref[...] + beta_ref[...]
    return jnp.maximum(hb, 0.0) * _row_mask(gidx)


def _apply_body(o_ref, sums_ref, dis_ref, g_ref, beta_ref, w_ref, fcw_ref, y_ref,
                hn_ref, u_ref, yn_ref):
    gidx = pl.program_id(0)
    hb = _bn_relu_block(o_ref, sums_ref, g_ref, beta_ref, gidx)
    hw = jnp.dot(hb, w_ref[...], preferred_element_type=jnp.float32, precision=_HI)
    hn_ref[...] = hw
    u_ref[...] = dis_ref[...] * hw
    yn_ref[...] = y_ref[...] + jnp.dot(hb, fcw_ref[...], preferred_element_type=jnp.float32, precision=_HI)


@jax.jit
def _tc_apply(o, sums, dis, g, beta, W_next, fcW_next, y):
    return pl.pallas_call(
        _apply_body,
        grid=(GB,),
        in_specs=[
            pl.BlockSpec((R, DIM), lambda g: (g, 0)),
            pl.BlockSpec((8, DIM), lambda g: (0, 0)),
            pl.BlockSpec((R, 1), lambda g: (g, 0)),
            pl.BlockSpec((1, DIM), lambda g: (0, 0)),
            pl.BlockSpec((1, DIM), lambda g: (0, 0)),
            pl.BlockSpec((DIM, DIM), lambda g: (0, 0)),
            pl.BlockSpec((DIM, C), lambda g: (0, 0)),
            pl.BlockSpec((R, C), lambda g: (g, 0)),
        ],
        out_specs=[
            pl.BlockSpec((R, DIM), lambda g: (g, 0)),
            pl.BlockSpec((R, DIM), lambda g: (g, 0)),
            pl.BlockSpec((R, C), lambda g: (g, 0)),
        ],
        out_shape=[
            jax.ShapeDtypeStruct((NPAD, DIM), jnp.float32),
            jax.ShapeDtypeStruct((NPAD, DIM), jnp.float32),
            jax.ShapeDtypeStruct((NPAD, C), jnp.float32),
        ],
    )(o, sums, dis, g, beta, W_next, fcW_next, y)


def _final_body(o_ref, sums_ref, g_ref, beta_ref, fcw_ref, y_ref, out_ref):
    gidx = pl.program_id(0)
    hb = _bn_relu_block(o_ref, sums_ref, g_ref, beta_ref, gidx)
    y = y_ref[...] + jnp.dot(hb, fcw_ref[...], preferred_element_type=jnp.float32, precision=_HI)
    z = y - jnp.max(y, axis=-1, keepdims=True)
    out_ref[...] = z - jnp.log(jnp.sum(jnp.exp(z), axis=-1, keepdims=True))


@jax.jit
def _tc_final(o, sums, g, beta, fcW, y):
    return pl.pallas_call(
        _final_body,
        grid=(GB,),
        in_specs=[
            pl.BlockSpec((R, DIM), lambda g: (g, 0)),
            pl.BlockSpec((8, DIM), lambda g: (0, 0)),
            pl.BlockSpec((1, DIM), lambda g: (0, 0)),
            pl.BlockSpec((1, DIM), lambda g: (0, 0)),
            pl.BlockSpec((DIM, C), lambda g: (0, 0)),
            pl.BlockSpec((R, C), lambda g: (g, 0)),
        ],
        out_specs=pl.BlockSpec((R, C), lambda g: (g, 0)),
        out_shape=jax.ShapeDtypeStruct((NPAD, C), jnp.float32),
    )(o, sums, g, beta, fcW, y)


def kernel(x, edge_index, W0, b0, W1, b1, W2, b2, W3, b3,
           g0, beta0, g1, beta1, g2, beta2, g3, beta3,
           fcW0, fcb0, fcW1, fcb1, fcW2, fcb2, fcW3, fcb3, fcW4, fcb4):
    ei = edge_index.astype(jnp.int32)
    pad = jnp.full((EPAD - E,), DUMMY, jnp.int32)
    src = jnp.concatenate([ei[0], pad])
    dst = jnp.concatenate([ei[1], pad])
    xp = jnp.concatenate([x, jnp.zeros((NPAD - N, F_IN), jnp.float32)], axis=0)

    fcb_sum = (fcb0 + fcb1 + fcb2 + fcb3 + fcb4).reshape(1, C)
    Ws = [W1, W2, W3]
    bs = [b0.reshape(1, DIM), b1.reshape(1, DIM), b2.reshape(1, DIM), b3.reshape(1, DIM)]
    gs = [g0.reshape(1, DIM), g1.reshape(1, DIM), g2.reshape(1, DIM), g3.reshape(1, DIM)]
    betas = [beta0.reshape(1, DIM), beta1.reshape(1, DIM), beta2.reshape(1, DIM), beta3.reshape(1, DIM)]
    fcWs = [fcW1, fcW2, fcW3, fcW4]

    cnt = _sc_count(dst)
    dis, h, u, y = _tc_prep(xp, cnt, W0, fcW0, fcb_sum)
    for i in range(3):
        parts = _sc_agg(u, src, dst)
        o, sums = _tc_stats(parts, h, dis, bs[i])
        h, u, y = _tc_apply(o, sums, dis, gs[i], betas[i], Ws[i], fcWs[i], y)
    parts = _sc_agg(u, src, dst)
    o, sums = _tc_stats(parts, h, dis, bs[3])
    out = _tc_final(o, sums, gs[3], betas[3], fcWs[3], y)
    return out[:N]


# speedup vs baseline: 1.5600x; 1.2930x over previous
import functools

import jax
import jax.numpy as jnp
from jax import lax
from jax.experimental import pallas as pl
from jax.experimental.pallas import tpu as pltpu
from jax.experimental.pallas import tpu_sc as plsc

N = 10000
E = 320000
F_IN = 128
DIM = 128
C = 64
EPS = 1e-5

NC = 2
NS = 16
NW = NC * NS
K = 128

NPAD = 10112
ROWS_PER_SC = NPAD // NS
DUMMY = 10008
NCH = 79
EPAD = NW * NCH * K

_ZR = 64
_NBUF = 4


def _sc_mesh():
    return plsc.VectorSubcoreMesh(
        core_axis_name="c", subcore_axis_name="s", num_cores=NC, num_subcores=NS
    )


def _fill_2d(ref, rows, width, value):
    v = jnp.full((16,), value, jnp.float32)

    def body(i, _):
        def inner(j, _):
            ref[i, pl.ds(j * 16, 16)] = v
            return 0

        return lax.fori_loop(0, width // 16, inner, 0)

    lax.fori_loop(0, rows, body, 0)


def _zero_shared(zbuf, acc, row0, nrows, width):
    nfull = nrows // _ZR
    rem = nrows - nfull * _ZR

    def body(i, _):
        pltpu.sync_copy(zbuf, acc.at[pl.ds(row0 + i * _ZR, _ZR), :])
        return 0

    lax.fori_loop(0, nfull, body, 0)
    if rem:
        pltpu.sync_copy(
            zbuf.at[pl.ds(0, rem), :], acc.at[pl.ds(row0 + nfull * _ZR, rem), :]
        )


def _count_body(dst_hbm, out_hbm, dst_v, ones_v, zbuf_v, acc_sh, sem):
    c = lax.axis_index("c")
    s = lax.axis_index("s")
    wid = c * NS + s
    _fill_2d(zbuf_v, _ZR, DIM, 0.0)
    _fill_2d(ones_v, K, DIM, 1.0)
    _zero_shared(zbuf_v, acc_sh, s * ROWS_PER_SC, ROWS_PER_SC, DIM)
    plsc.subcore_barrier()

    base = wid * NCH * K

    def chunk(t, _):
        pltpu.sync_copy(dst_hbm.at[pl.ds(base + t * K, K)], dst_v)
        pltpu.sync_copy(ones_v, acc_sh.at[dst_v], add=True)
        return 0

    lax.fori_loop(0, NCH, chunk, 0)
    plsc.subcore_barrier()
    pltpu.sync_copy(
        acc_sh.at[pl.ds(s * ROWS_PER_SC, ROWS_PER_SC), :],
        out_hbm.at[c, pl.ds(s * ROWS_PER_SC, ROWS_PER_SC), :],
    )


def _agg_body(u_hbm, src_hbm, dst_hbm, out_hbm, src_v, dst_v,
              rows_v, zbuf_v, acc_sh, sem):
    c = lax.axis_index("c")
    s = lax.axis_index("s")
    wid = c * NS + s
    _fill_2d(zbuf_v, _ZR, DIM, 0.0)
    _zero_shared(zbuf_v, acc_sh, s * ROWS_PER_SC, ROWS_PER_SC, DIM)
    plsc.subcore_barrier()

    base = wid * NCH * K

    def body(t, _):
        off = base + t * K
        pltpu.sync_copy(src_hbm.at[pl.ds(off, K)], src_v)
        pltpu.sync_copy(dst_hbm.at[pl.ds(off, K)], dst_v)
        pltpu.async_copy(u_hbm.at[src_v], rows_v, sem).wait()
        pltpu.sync_copy(rows_v, acc_sh.at[dst_v], add=True)
        return 0

    lax.fori_loop(0, NCH, body, 0)
    plsc.subcore_barrier()
    pltpu.sync_copy(
        acc_sh.at[pl.ds(s * ROWS_PER_SC, ROWS_PER_SC), :],
        out_hbm.at[c, pl.ds(s * ROWS_PER_SC, ROWS_PER_SC), :],
    )


@jax.jit
def _sc_count(dst):
    return pl.kernel(
        _count_body,
        out_type=jax.ShapeDtypeStruct((NC, NPAD, DIM), jnp.float32),
        mesh=_sc_mesh(),
        scratch_types=[
            pltpu.VMEM((K,), jnp.int32),
            pltpu.VMEM((K, DIM), jnp.float32),
            pltpu.VMEM((_ZR, DIM), jnp.float32),
            pltpu.VMEM_SHARED((NPAD, DIM), jnp.float32),
            pltpu.SemaphoreType.DMA,
        ],
    )(dst)


@jax.jit
def _sc_agg(u, src, dst):
    return pl.kernel(
        _agg_body,
        out_type=jax.ShapeDtypeStruct((NC, NPAD, DIM), jnp.float32),
        mesh=_sc_mesh(),
        scratch_types=[
            pltpu.VMEM((K,), jnp.int32),
            pltpu.VMEM((K,), jnp.int32),
            pltpu.VMEM((K, DIM), jnp.float32),
            pltpu.VMEM((_ZR, DIM), jnp.float32),
            pltpu.VMEM_SHARED((NPAD, DIM), jnp.float32),
            pltpu.SemaphoreType.DMA,
        ],
    )(u, src, dst)


GB = 8
R = NPAD // GB
_HI = lax.Precision.HIGHEST


def _row_mask(g):
    rowid = lax.broadcasted_iota(jnp.int32, (R, 1), 0) + g * R
    return jnp.where(rowid < N, 1.0, 0.0).astype(jnp.float32)


def _prep_body(x_ref, cnt_ref, w0_ref, fcw0_ref, fcb_ref, dis_ref, h_ref, u_ref, y_ref):
    cnt = cnt_ref[0, :, 0:1] + cnt_ref[1, :, 0:1]
    dis = lax.rsqrt(cnt + 2.0)
    x = x_ref[...]
    h = jnp.dot(x, w0_ref[...], preferred_element_type=jnp.float32, precision=_HI)
    dis_ref[...] = dis
    h_ref[...] = h
    u_ref[...] = dis * h
    y_ref[...] = jnp.dot(x, fcw0_ref[...], preferred_element_type=jnp.float32, precision=_HI) + fcb_ref[...]


@jax.jit
def _tc_prep(x, cnt, W0, fcW0, fcb_sum):
    return pl.pallas_call(
        _prep_body,
        grid=(GB,),
        in_specs=[
            pl.BlockSpec((R, DIM), lambda g: (g, 0)),
            pl.BlockSpec((NC, R, DIM), lambda g: (0, g, 0)),
            pl.BlockSpec((DIM, DIM), lambda g: (0, 0)),
            pl.BlockSpec((DIM, C), lambda g: (0, 0)),
            pl.BlockSpec((1, C), lambda g: (0, 0)),
        ],
        out_specs=[
            pl.BlockSpec((R, 1), lambda g: (g, 0)),
            pl.BlockSpec((R, DIM), lambda g: (g, 0)),
            pl.BlockSpec((R, DIM), lambda g: (g, 0)),
            pl.BlockSpec((R, C), lambda g: (g, 0)),
        ],
        out_shape=[
            jax.ShapeDtypeStruct((NPAD, 1), jnp.float32),
            jax.ShapeDtypeStruct((NPAD, DIM), jnp.float32),
            jax.ShapeDtypeStruct((NPAD, DIM), jnp.float32),
            jax.ShapeDtypeStruct((NPAD, C), jnp.float32),
        ],
    )(x, cnt, W0, fcW0, fcb_sum)


def _stats_body(parts_ref, h_ref, dis_ref, b_ref, o_ref, sums_ref):
    g = pl.program_id(0)
    dis = dis_ref[...]
    tmp = parts_ref[0] + parts_ref[1]
    o = (dis * tmp + (2.0 * dis * dis) * h_ref[...] + b_ref[...]) * _row_mask(g)
    o_ref[...] = o
    s0 = jnp.sum(o, axis=0, keepdims=True)
    s1 = jnp.sum(o * o, axis=0, keepdims=True)
    blk = jnp.concatenate([s0, s1, jnp.zeros((6, DIM), jnp.float32)], axis=0)

    @pl.when(g == 0)
    def _():
        sums_ref[...] = blk

    @pl.when(g != 0)
    def _():
        sums_ref[...] = sums_ref[...] + blk


@jax.jit
def _tc_stats(parts, h, dis, b):
    return pl.pallas_call(
        _stats_body,
        grid=(GB,),
        in_specs=[
            pl.BlockSpec((NC, R, DIM), lambda g: (0, g, 0)),
            pl.BlockSpec((R, DIM), lambda g: (g, 0)),
            pl.BlockSpec((R, 1), lambda g: (g, 0)),
            pl.BlockSpec((1, DIM), lambda g: (0, 0)),
        ],
        out_specs=[
            pl.BlockSpec((R, DIM), lambda g: (g, 0)),
            pl.BlockSpec((8, DIM), lambda g: (0, 0)),
        ],
        out_shape=[
            jax.ShapeDtypeStruct((NPAD, DIM), jnp.float32),
            jax.ShapeDtypeStruct((8, DIM), jnp.float32),
        ],
    )(parts, h, dis, b)


def _bn_relu_block(o_ref, sums_ref, g_ref, beta_ref, gidx):
    m = sums_ref[0:1, :] * (1.0 / N)
    s2 = sums_ref[1:2, :] * (1.0 / N)
    v = s2 - m * m
    hb = (o_ref[...] - m) * lax.rsqrt(v + EPS) * g_ref[...] + beta_ref[...]
    return jnp.maximum(hb, 0.0) * _row_mask(gidx)


def _apply_body(o_ref, sums_ref, dis_ref, g_ref, beta_ref, w_ref, fcw_ref, y_ref,
                hn_ref, u_ref, yn_ref):
    gidx = pl.program_id(0)
    hb = _bn_relu_block(o_ref, sums_ref, g_ref, beta_ref, gidx)
    hw = jnp.dot(hb, w_ref[...], preferred_element_type=jnp.float32, precision=_HI)
    hn_ref[...] = hw
    u_ref[...] = dis_ref[...] * hw
    yn_ref[...] = y_ref[...] + jnp.dot(hb, fcw_ref[...], preferred_element_type=jnp.float32, precision=_HI)


@jax.jit
def _tc_apply(o, sums, dis, g, beta, W_next, fcW_next, y):
    return pl.pallas_call(
        _apply_body,
        grid=(GB,),
        in_specs=[
            pl.BlockSpec((R, DIM), lambda g: (g, 0)),
            pl.BlockSpec((8, DIM), lambda g: (0, 0)),
            pl.BlockSpec((R, 1), lambda g: (g, 0)),
            pl.BlockSpec((1, DIM), lambda g: (0, 0)),
            pl.BlockSpec((1, DIM), lambda g: (0, 0)),
            pl.BlockSpec((DIM, DIM), lambda g: (0, 0)),
            pl.BlockSpec((DIM, C), lambda g: (0, 0)),
            pl.BlockSpec((R, C), lambda g: (g, 0)),
        ],
        out_specs=[
            pl.BlockSpec((R, DIM), lambda g: (g, 0)),
            pl.BlockSpec((R, DIM), lambda g: (g, 0)),
            pl.BlockSpec((R, C), lambda g: (g, 0)),
        ],
        out_shape=[
            jax.ShapeDtypeStruct((NPAD, DIM), jnp.float32),
            jax.ShapeDtypeStruct((NPAD, DIM), jnp.float32),
            jax.ShapeDtypeStruct((NPAD, C), jnp.float32),
        ],
    )(o, sums, dis, g, beta, W_next, fcW_next, y)


def _final_body(o_ref, sums_ref, g_ref, beta_ref, fcw_ref, y_ref, out_ref):
    gidx = pl.program_id(0)
    hb = _bn_relu_block(o_ref, sums_ref, g_ref, beta_ref, gidx)
    y = y_ref[...] + jnp.dot(hb, fcw_ref[...], preferred_element_type=jnp.float32, precision=_HI)
    z = y - jnp.max(y, axis=-1, keepdims=True)
    out_ref[...] = z - jnp.log(jnp.sum(jnp.exp(z), axis=-1, keepdims=True))


@jax.jit
def _tc_final(o, sums, g, beta, fcW, y):
    return pl.pallas_call(
        _final_body,
        grid=(GB,),
        in_specs=[
            pl.BlockSpec((R, DIM), lambda g: (g, 0)),
            pl.BlockSpec((8, DIM), lambda g: (0, 0)),
            pl.BlockSpec((1, DIM), lambda g: (0, 0)),
            pl.BlockSpec((1, DIM), lambda g: (0, 0)),
            pl.BlockSpec((DIM, C), lambda g: (0, 0)),
            pl.BlockSpec((R, C), lambda g: (g, 0)),
        ],
        out_specs=pl.BlockSpec((R, C), lambda g: (g, 0)),
        out_shape=jax.ShapeDtypeStruct((NPAD, C), jnp.float32),
    )(o, sums, g, beta, fcW, y)


def kernel(x, edge_index, W0, b0, W1, b1, W2, b2, W3, b3,
           g0, beta0, g1, beta1, g2, beta2, g3, beta3,
           fcW0, fcb0, fcW1, fcb1, fcW2, fcb2, fcW3, fcb3, fcW4, fcb4):
    ei = edge_index.astype(jnp.int32)
    pad = jnp.full((EPAD - E,), DUMMY, jnp.int32)
    src = jnp.concatenate([ei[0], pad])
    dst = jnp.concatenate([ei[1], pad])
    xp = jnp.concatenate([x, jnp.zeros((NPAD - N, F_IN), jnp.float32)], axis=0)

    fcb_sum = (fcb0 + fcb1 + fcb2 + fcb3 + fcb4).reshape(1, C)
    Ws = [W1, W2, W3]
    bs = [b0.reshape(1, DIM), b1.reshape(1, DIM), b2.reshape(1, DIM), b3.reshape(1, DIM)]
    gs = [g0.reshape(1, DIM), g1.reshape(1, DIM), g2.reshape(1, DIM), g3.reshape(1, DIM)]
    betas = [beta0.reshape(1, DIM), beta1.reshape(1, DIM), beta2.reshape(1, DIM), beta3.reshape(1, DIM)]
    fcWs = [fcW1, fcW2, fcW3, fcW4]

    cnt = _sc_count(dst)
    dis, h, u, y = _tc_prep(xp, cnt, W0, fcW0, fcb_sum)
    for i in range(3):
        parts = _sc_agg(u, src, dst)
        o, sums = _tc_stats(parts, h, dis, bs[i])
        h, u, y = _tc_apply(o, sums, dis, gs[i], betas[i], Ws[i], fcWs[i], y)
    parts = _sc_agg(u, src, dst)
    o, sums = _tc_stats(parts, h, dis, bs[3])
    out = _tc_final(o, sums, gs[3], betas[3], fcWs[3], y)
    return out[:N]


# speedup vs baseline: 1.6671x; 1.0687x over previous
import functools

import jax
import jax.numpy as jnp
from jax import lax
from jax.experimental import pallas as pl
from jax.experimental.pallas import tpu as pltpu
from jax.experimental.pallas import tpu_sc as plsc

N = 10000
E = 320000
F_IN = 128
DIM = 128
C = 64
EPS = 1e-5

NC = 2
NS = 16
NW = NC * NS
K = 128

NPAD = 10112
ROWS_PER_SC = NPAD // NS
DUMMY = 10008
NCH = 79
EPAD = NW * NCH * K

_ZR = 64
_NBUF = 4


def _sc_mesh():
    return plsc.VectorSubcoreMesh(
        core_axis_name="c", subcore_axis_name="s", num_cores=NC, num_subcores=NS
    )


def _fill_2d(ref, rows, width, value):
    v = jnp.full((16,), value, jnp.float32)

    def body(i, _):
        def inner(j, _):
            ref[i, pl.ds(j * 16, 16)] = v
            return 0

        return lax.fori_loop(0, width // 16, inner, 0)

    lax.fori_loop(0, rows, body, 0)


def _zero_shared(zbuf, acc, row0, nrows, width):
    nfull = nrows // _ZR
    rem = nrows - nfull * _ZR

    def body(i, _):
        pltpu.sync_copy(zbuf, acc.at[pl.ds(row0 + i * _ZR, _ZR), :])
        return 0

    lax.fori_loop(0, nfull, body, 0)
    if rem:
        pltpu.sync_copy(
            zbuf.at[pl.ds(0, rem), :], acc.at[pl.ds(row0 + nfull * _ZR, rem), :]
        )


def _count_body(dst_hbm, out_hbm, dst_v, ones_v, zbuf_v, acc_sh, sem):
    c = lax.axis_index("c")
    s = lax.axis_index("s")
    wid = c * NS + s
    _fill_2d(zbuf_v, _ZR, DIM, 0.0)
    _fill_2d(ones_v, K, DIM, 1.0)
    _zero_shared(zbuf_v, acc_sh, s * ROWS_PER_SC, ROWS_PER_SC, DIM)
    plsc.subcore_barrier()

    base = wid * NCH * K

    def chunk(t, _):
        pltpu.sync_copy(dst_hbm.at[pl.ds(base + t * K, K)], dst_v)
        pltpu.sync_copy(ones_v, acc_sh.at[dst_v], add=True)
        return 0

    lax.fori_loop(0, NCH, chunk, 0)
    plsc.subcore_barrier()
    pltpu.sync_copy(
        acc_sh.at[pl.ds(s * ROWS_PER_SC, ROWS_PER_SC), :],
        out_hbm.at[c, pl.ds(s * ROWS_PER_SC, ROWS_PER_SC), :],
    )


def _agg_body(u_hbm, sd_hbm, out_hbm, sd_v, rows_v, zbuf_v, acc_sh, sem):
    c = lax.axis_index("c")
    s = lax.axis_index("s")
    wid = c * NS + s
    _fill_2d(zbuf_v, _ZR, DIM, 0.0)
    _zero_shared(zbuf_v, acc_sh, s * ROWS_PER_SC, ROWS_PER_SC, DIM)
    plsc.subcore_barrier()

    base = wid * NCH

    def body(t, _):
        pltpu.sync_copy(sd_hbm.at[base + t], sd_v)
        pltpu.async_copy(u_hbm.at[sd_v.at[0]], rows_v, sem).wait()
        pltpu.sync_copy(rows_v, acc_sh.at[sd_v.at[1]], add=True)
        return 0

    lax.fori_loop(0, NCH, body, 0)
    plsc.subcore_barrier()
    pltpu.sync_copy(
        acc_sh.at[pl.ds(s * ROWS_PER_SC, ROWS_PER_SC), :],
        out_hbm.at[c, pl.ds(s * ROWS_PER_SC, ROWS_PER_SC), :],
    )


@jax.jit
def _sc_count(dst):
    return pl.kernel(
        _count_body,
        out_type=jax.ShapeDtypeStruct((NC, NPAD, DIM), jnp.float32),
        mesh=_sc_mesh(),
        scratch_types=[
            pltpu.VMEM((K,), jnp.int32),
            pltpu.VMEM((K, DIM), jnp.float32),
            pltpu.VMEM((_ZR, DIM), jnp.float32),
            pltpu.VMEM_SHARED((NPAD, DIM), jnp.float32),
            pltpu.SemaphoreType.DMA,
        ],
    )(dst)


@jax.jit
def _sc_agg(u, sd):
    return pl.kernel(
        _agg_body,
        out_type=jax.ShapeDtypeStruct((NC, NPAD, DIM), jnp.float32),
        mesh=_sc_mesh(),
        scratch_types=[
            pltpu.VMEM((2, K), jnp.int32),
            pltpu.VMEM((K, DIM), jnp.float32),
            pltpu.VMEM((_ZR, DIM), jnp.float32),
            pltpu.VMEM_SHARED((NPAD, DIM), jnp.float32),
            pltpu.SemaphoreType.DMA,
        ],
    )(u, sd)


GB = 8
R = NPAD // GB
_HI = lax.Precision.HIGHEST


def _row_mask(g):
    rowid = lax.broadcasted_iota(jnp.int32, (R, 1), 0) + g * R
    return jnp.where(rowid < N, 1.0, 0.0).astype(jnp.float32)


def _prep_body(x_ref, cnt_ref, w0_ref, fcw0_ref, fcb_ref, dis_ref, h_ref, u_ref, y_ref):
    cnt = cnt_ref[0, :, 0:1] + cnt_ref[1, :, 0:1]
    dis = lax.rsqrt(cnt + 2.0)
    x = x_ref[...]
    h = jnp.dot(x, w0_ref[...], preferred_element_type=jnp.float32, precision=_HI)
    dis_ref[...] = dis
    h_ref[...] = h
    u_ref[...] = dis * h
    y_ref[...] = jnp.dot(x, fcw0_ref[...], preferred_element_type=jnp.float32, precision=_HI) + fcb_ref[...]


@jax.jit
def _tc_prep(x, cnt, W0, fcW0, fcb_sum):
    return pl.pallas_call(
        _prep_body,
        grid=(GB,),
        in_specs=[
            pl.BlockSpec((R, DIM), lambda g: (g, 0)),
            pl.BlockSpec((NC, R, DIM), lambda g: (0, g, 0)),
            pl.BlockSpec((DIM, DIM), lambda g: (0, 0)),
            pl.BlockSpec((DIM, C), lambda g: (0, 0)),
            pl.BlockSpec((1, C), lambda g: (0, 0)),
        ],
        out_specs=[
            pl.BlockSpec((R, 1), lambda g: (g, 0)),
            pl.BlockSpec((R, DIM), lambda g: (g, 0)),
            pl.BlockSpec((R, DIM), lambda g: (g, 0)),
            pl.BlockSpec((R, C), lambda g: (g, 0)),
        ],
        out_shape=[
            jax.ShapeDtypeStruct((NPAD, 1), jnp.float32),
            jax.ShapeDtypeStruct((NPAD, DIM), jnp.float32),
            jax.ShapeDtypeStruct((NPAD, DIM), jnp.float32),
            jax.ShapeDtypeStruct((NPAD, C), jnp.float32),
        ],
    )(x, cnt, W0, fcW0, fcb_sum)


def _stats_body(parts_ref, h_ref, dis_ref, b_ref, o_ref, sums_ref):
    g = pl.program_id(0)
    dis = dis_ref[...]
    tmp = parts_ref[0] + parts_ref[1]
    o = (dis * tmp + (2.0 * dis * dis) * h_ref[...] + b_ref[...]) * _row_mask(g)
    o_ref[...] = o
    s0 = jnp.sum(o, axis=0, keepdims=True)
    s1 = jnp.sum(o * o, axis=0, keepdims=True)
    blk = jnp.concatenate([s0, s1, jnp.zeros((6, DIM), jnp.float32)], axis=0)

    @pl.when(g == 0)
    def _():
        sums_ref[...] = blk

    @pl.when(g != 0)
    def _():
        sums_ref[...] = sums_ref[...] + blk


@jax.jit
def _tc_stats(parts, h, dis, b):
    return pl.pallas_call(
        _stats_body,
        grid=(GB,),
        in_specs=[
            pl.BlockSpec((NC, R, DIM), lambda g: (0, g, 0)),
            pl.BlockSpec((R, DIM), lambda g: (g, 0)),
            pl.BlockSpec((R, 1), lambda g: (g, 0)),
            pl.BlockSpec((1, DIM), lambda g: (0, 0)),
        ],
        out_specs=[
            pl.BlockSpec((R, DIM), lambda g: (g, 0)),
            pl.BlockSpec((8, DIM), lambda g: (0, 0)),
        ],
        out_shape=[
            jax.ShapeDtypeStruct((NPAD, DIM), jnp.float32),
            jax.ShapeDtypeStruct((8, DIM), jnp.float32),
        ],
    )(parts, h, dis, b)


def _bn_relu_block(o_ref, sums_ref, g_ref, beta_ref, gidx):
    m = sums_ref[0:1, :] * (1.0 / N)
    s2 = sums_ref[1:2, :] * (1.0 / N)
    v = s2 - m * m
    hb = (o_ref[...] - m) * lax.rsqrt(v + EPS) * g_ref[...] + beta_ref[...]
    return jnp.maximum(hb, 0.0) * _row_mask(gidx)


def _apply_body(o_ref, sums_ref, dis_ref, g_ref, beta_ref, w_ref, fcw_ref, y_ref,
                hn_ref, u_ref, yn_ref):
    gidx = pl.program_id(0)
    hb = _bn_relu_block(o_ref, sums_ref, g_ref, beta_ref, gidx)
    hw = jnp.dot(hb, w_ref[...], preferred_element_type=jnp.float32, precision=_HI)
    hn_ref[...] = hw
    u_ref[...] = dis_ref[...] * hw
    yn_ref[...] = y_ref[...] + jnp.dot(hb, fcw_ref[...], preferred_element_type=jnp.float32, precision=_HI)


@jax.jit
def _tc_apply(o, sums, dis, g, beta, W_next, fcW_next, y):
    return pl.pallas_call(
        _apply_body,
        grid=(GB,),
        in_specs=[
            pl.BlockSpec((R, DIM), lambda g: (g, 0)),
            pl.BlockSpec((8, DIM), lambda g: (0, 0)),
            pl.BlockSpec((R, 1), lambda g: (g, 0)),
            pl.BlockSpec((1, DIM), lambda g: (0, 0)),
            pl.BlockSpec((1, DIM), lambda g: (0, 0)),
            pl.BlockSpec((DIM, DIM), lambda g: (0, 0)),
            pl.BlockSpec((DIM, C), lambda g: (0, 0)),
            pl.BlockSpec((R, C), lambda g: (g, 0)),
        ],
        out_specs=[
            pl.BlockSpec((R, DIM), lambda g: (g, 0)),
            pl.BlockSpec((R, DIM), lambda g: (g, 0)),
            pl.BlockSpec((R, C), lambda g: (g, 0)),
        ],
        out_shape=[
            jax.ShapeDtypeStruct((NPAD, DIM), jnp.float32),
            jax.ShapeDtypeStruct((NPAD, DIM), jnp.float32),
            jax.ShapeDtypeStruct((NPAD, C), jnp.float32),
        ],
    )(o, sums, dis, g, beta, W_next, fcW_next, y)


def _final_body(o_ref, sums_ref, g_ref, beta_ref, fcw_ref, y_ref, out_ref):
    gidx = pl.program_id(0)
    hb = _bn_relu_block(o_ref, sums_ref, g_ref, beta_ref, gidx)
    y = y_ref[...] + jnp.dot(hb, fcw_ref[...], preferred_element_type=jnp.float32, precision=_HI)
    z = y - jnp.max(y, axis=-1, keepdims=True)
    out_ref[...] = z - jnp.log(jnp.sum(jnp.exp(z), axis=-1, keepdims=True))


@jax.jit
def _tc_final(o, sums, g, beta, fcW, y):
    return pl.pallas_call(
        _final_body,
        grid=(GB,),
        in_specs=[
            pl.BlockSpec((R, DIM), lambda g: (g, 0)),
            pl.BlockSpec((8, DIM), lambda g: (0, 0)),
            pl.BlockSpec((1, DIM), lambda g: (0, 0)),
            pl.BlockSpec((1, DIM), lambda g: (0, 0)),
            pl.BlockSpec((DIM, C), lambda g: (0, 0)),
            pl.BlockSpec((R, C), lambda g: (g, 0)),
        ],
        out_specs=pl.BlockSpec((R, C), lambda g: (g, 0)),
        out_shape=jax.ShapeDtypeStruct((NPAD, C), jnp.float32),
    )(o, sums, g, beta, fcW, y)


def kernel(x, edge_index, W0, b0, W1, b1, W2, b2, W3, b3,
           g0, beta0, g1, beta1, g2, beta2, g3, beta3,
           fcW0, fcb0, fcW1, fcb1, fcW2, fcb2, fcW3, fcb3, fcW4, fcb4):
    ei = edge_index.astype(jnp.int32)
    pad = jnp.full((EPAD - E,), DUMMY, jnp.int32)
    src = jnp.concatenate([ei[0], pad])
    dst = jnp.concatenate([ei[1], pad])
    sd = jnp.stack([src.reshape(NW * NCH, K), dst.reshape(NW * NCH, K)], axis=1)
    xp = jnp.concatenate([x, jnp.zeros((NPAD - N, F_IN), jnp.float32)], axis=0)

    fcb_sum = (fcb0 + fcb1 + fcb2 + fcb3 + fcb4).reshape(1, C)
    Ws = [W1, W2, W3]
    bs = [b0.reshape(1, DIM), b1.reshape(1, DIM), b2.reshape(1, DIM), b3.reshape(1, DIM)]
    gs = [g0.reshape(1, DIM), g1.reshape(1, DIM), g2.reshape(1, DIM), g3.reshape(1, DIM)]
    betas = [beta0.reshape(1, DIM), beta1.reshape(1, DIM), beta2.reshape(1, DIM), beta3.reshape(1, DIM)]
    fcWs = [fcW1, fcW2, fcW3, fcW4]

    cnt = _sc_count(dst)
    dis, h, u, y = _tc_prep(xp, cnt, W0, fcW0, fcb_sum)
    for i in range(3):
        parts = _sc_agg(u, sd)
        o, sums = _tc_stats(parts, h, dis, bs[i])
        h, u, y = _tc_apply(o, sums, dis, gs[i], betas[i], Ws[i], fcWs[i], y)
    parts = _sc_agg(u, sd)
    o, sums = _tc_stats(parts, h, dis, bs[3])
    out = _tc_final(o, sums, gs[3], betas[3], fcWs[3], y)
    return out[:N]
